# baseline (speedup 1.0000x reference)
import functools

import jax
import jax.numpy as jnp
from jax.experimental import pallas as pl
from jax.experimental.pallas import tpu as pltpu

_N_SAMPLE = 16


def _agg_kernel(x_ref, nb_ref, e_ref, wxn_ref, wn_ref, me_ref, wm_ref,
                b_ref, out_ref):
    d_hid = wn_ref.shape[0]

    hx = jax.lax.dot_general(
        x_ref[...].astype(jnp.bfloat16), wxn_ref[...],
        (((1,), (1,)), ((), ())), preferred_element_type=jnp.float32)

    eb = e_ref[...].astype(jnp.bfloat16)

    acc = None
    for s in range(_N_SAMPLE):
        lo = s * d_hid
        nb_s = nb_ref[:, lo:lo + d_hid].astype(jnp.bfloat16)
        hn_s = jax.lax.dot_general(
            nb_s, wn_ref[...],
            (((1,), (1,)), ((), ())), preferred_element_type=jnp.float32)
        he_s = jax.lax.dot_general(
            eb, me_ref[:, lo:lo + d_hid],
            (((1,), (0,)), ((), ())), preferred_element_type=jnp.float32)
        a_s = hn_s + he_s
        acc = a_s if acc is None else jnp.maximum(acc, a_s)

    m = jax.nn.relu(acc + hx)

    out_ref[...] = jax.lax.dot_general(
        m.astype(jnp.bfloat16), wm_ref[...],
        (((1,), (1,)), ((), ())), preferred_element_type=jnp.float32
    ) + b_ref[...]


@functools.partial(jax.jit, static_argnames=("block_n",))
def _run(x, neibs2, e2, wxn, wn, me, wm, b2d, block_n):
    n, d_in = x.shape
    d_hid = wn.shape[0]
    d_out = wm.shape[0]
    grid = (n // block_n,)

    return pl.pallas_call(
        _agg_kernel,
        grid=grid,
        in_specs=[
            pl.BlockSpec((block_n, d_in), lambda i: (i, 0)),
            pl.BlockSpec((block_n, _N_SAMPLE * d_in), lambda i: (i, 0)),
            pl.BlockSpec((block_n, e2.shape[1]), lambda i: (i, 0)),
            pl.BlockSpec((d_hid, d_in), lambda i: (0, 0)),
            pl.BlockSpec((d_hid, d_in), lambda i: (0, 0)),
            pl.BlockSpec((e2.shape[1], _N_SAMPLE * d_hid), lambda i: (0, 0)),
            pl.BlockSpec((d_out, d_hid), lambda i: (0, 0)),
            pl.BlockSpec((1, d_out), lambda i: (0, 0)),
        ],
        out_specs=pl.BlockSpec((block_n, d_out), lambda i: (i, 0)),
        out_shape=jax.ShapeDtypeStruct((n, d_out), jnp.float32),
        compiler_params=pltpu.CompilerParams(
            dimension_semantics=("parallel",)),
    )(x, neibs2, e2, wxn, wn, me, wm, b2d)


def kernel(x, neibs, edge_emb, mask, W_x, W_n, W_e, W_m, b_m):
    del mask
    n = x.shape[0]
    d_hid, d_edge = W_e.shape
    neibs2 = neibs.reshape(n, -1)
    e2 = edge_emb.reshape(n, -1)
    wxn = (W_x - W_n).astype(jnp.bfloat16)
    wn = W_n.astype(jnp.bfloat16)
    wm = W_m.astype(jnp.bfloat16)
    me = (jnp.eye(_N_SAMPLE, dtype=W_e.dtype)[:, None, :, None]
          * W_e.T[None, :, None, :]).reshape(
              _N_SAMPLE * d_edge, _N_SAMPLE * d_hid).astype(jnp.bfloat16)
    b2d = b_m.reshape(1, -1)
    block_n = 400 if n % 400 == 0 else n
    return _run(x, neibs2, e2, wxn, wn, me, wm, b2d, block_n)


# speedup vs baseline: 2.4240x; 2.4240x over previous
import functools

import jax
import jax.numpy as jnp
from jax.experimental import pallas as pl
from jax.experimental.pallas import tpu as pltpu

_N_SAMPLE = 16


def _agg_kernel(x_ref, neibs_ref, e_hbm, wxn_ref, wn_ref, we_ref, wm_ref,
                b_ref, out_ref, e_buf, e_sem):
    i = pl.program_id(0)
    n_steps = pl.num_programs(0)
    bn = x_ref.shape[0]
    d_hid = wn_ref.shape[0]
    rows = e_buf.shape[1]

    def e_copy(idx):
        slot = jax.lax.rem(idx, 2)
        return pltpu.make_async_copy(
            e_hbm.at[pl.ds(idx * rows, rows), :],
            e_buf.at[slot],
            e_sem.at[slot])

    @pl.when(i == 0)
    def _():
        e_copy(i).start()

    @pl.when(i + 1 < n_steps)
    def _():
        e_copy(i + 1).start()

    hx = jax.lax.dot_general(
        x_ref[...].astype(jnp.bfloat16), wxn_ref[...],
        (((1,), (1,)), ((), ())), preferred_element_type=jnp.float32)

    hn = jax.lax.dot_general(
        neibs_ref[...].astype(jnp.bfloat16), wn_ref[...],
        (((1,), (1,)), ((), ())), preferred_element_type=jnp.float32)

    e_copy(i).wait()
    he = jax.lax.dot_general(
        e_buf[jax.lax.rem(i, 2)].astype(jnp.bfloat16), we_ref[...],
        (((1,), (1,)), ((), ())), preferred_element_type=jnp.float32)

    a = (hn + he).reshape(bn, _N_SAMPLE, d_hid)
    m = jax.nn.relu(jnp.max(a, axis=1) + hx)

    out_ref[...] = jax.lax.dot_general(
        m.astype(jnp.bfloat16), wm_ref[...],
        (((1,), (1,)), ((), ())), preferred_element_type=jnp.float32
    ) + b_ref[...]


@functools.partial(jax.jit, static_argnames=("block_n",))
def _run(x, neibs, edge_emb, wxn, wn, we, wm, b2d, block_n):
    n, d_in = x.shape
    d_edge = edge_emb.shape[1]
    d_hid = wn.shape[0]
    d_out = wm.shape[0]
    grid = (n // block_n,)

    return pl.pallas_call(
        _agg_kernel,
        grid=grid,
        in_specs=[
            pl.BlockSpec((block_n, d_in), lambda i: (i, 0)),
            pl.BlockSpec((block_n * _N_SAMPLE, d_in), lambda i: (i, 0)),
            pl.BlockSpec(memory_space=pltpu.MemorySpace.HBM),
            pl.BlockSpec((d_hid, d_in), lambda i: (0, 0)),
            pl.BlockSpec((d_hid, d_in), lambda i: (0, 0)),
            pl.BlockSpec((d_hid, d_edge), lambda i: (0, 0)),
            pl.BlockSpec((d_out, d_hid), lambda i: (0, 0)),
            pl.BlockSpec((1, d_out), lambda i: (0, 0)),
        ],
        out_specs=pl.BlockSpec((block_n, d_out), lambda i: (i, 0)),
        out_shape=jax.ShapeDtypeStruct((n, d_out), jnp.float32),
        scratch_shapes=[
            pltpu.VMEM((2, block_n * _N_SAMPLE, d_edge), jnp.float32),
            pltpu.SemaphoreType.DMA((2,)),
        ],
        compiler_params=pltpu.CompilerParams(
            dimension_semantics=("arbitrary",)),
    )(x, neibs, edge_emb, wxn, wn, we, wm, b2d)


def kernel(x, neibs, edge_emb, mask, W_x, W_n, W_e, W_m, b_m):
    del mask
    n = x.shape[0]
    wxn = (W_x - W_n).astype(jnp.bfloat16)
    wn = W_n.astype(jnp.bfloat16)
    we = W_e.astype(jnp.bfloat16)
    wm = W_m.astype(jnp.bfloat16)
    b2d = b_m.reshape(1, -1)
    block_n = 400 if n % 400 == 0 else n
    return _run(x, neibs, edge_emb, wxn, wn, we, wm, b2d, block_n)


# speedup vs baseline: 2.4277x; 1.0015x over previous
import functools

import jax
import jax.numpy as jnp
from jax.experimental import pallas as pl
from jax.experimental.pallas import tpu as pltpu

_N_SAMPLE = 16


def _agg_kernel(x_ref, neibs_ref, e_hbm, wxn_ref, wn_ref, we_ref, wm_ref,
                b_ref, out_ref, e_buf, e_sem):
    i = pl.program_id(0)
    n_steps = pl.num_programs(0)
    bn = x_ref.shape[0]
    d_hid = wn_ref.shape[0]
    rows = e_buf.shape[1]

    n_chunks = 4
    chunk = rows // n_chunks

    def e_copy(idx, c):
        slot = jax.lax.rem(idx, 2)
        return pltpu.make_async_copy(
            e_hbm.at[pl.ds(idx * rows + c * chunk, chunk), :],
            e_buf.at[slot, pl.ds(c * chunk, chunk), :],
            e_sem.at[slot, c])

    @pl.when(i == 0)
    def _():
        for c in range(n_chunks):
            e_copy(i, c).start()

    @pl.when(i + 1 < n_steps)
    def _():
        for c in range(n_chunks):
            e_copy(i + 1, c).start()

    hx = jax.lax.dot_general(
        x_ref[...].astype(jnp.bfloat16), wxn_ref[...],
        (((1,), (1,)), ((), ())), preferred_element_type=jnp.float32)

    hn = jax.lax.dot_general(
        neibs_ref[...].astype(jnp.bfloat16), wn_ref[...],
        (((1,), (1,)), ((), ())), preferred_element_type=jnp.float32)

    for c in range(n_chunks):
        e_copy(i, c).wait()
    he = jax.lax.dot_general(
        e_buf[jax.lax.rem(i, 2)].astype(jnp.bfloat16), we_ref[...],
        (((1,), (1,)), ((), ())), preferred_element_type=jnp.float32)

    a = (hn + he).reshape(bn, _N_SAMPLE, d_hid)
    m = jax.nn.relu(jnp.max(a, axis=1) + hx)

    out_ref[...] = jax.lax.dot_general(
        m.astype(jnp.bfloat16), wm_ref[...],
        (((1,), (1,)), ((), ())), preferred_element_type=jnp.float32
    ) + b_ref[...]


@functools.partial(jax.jit, static_argnames=("block_n",))
def _run(x, neibs, edge_emb, wxn, wn, we, wm, b2d, block_n):
    n, d_in = x.shape
    d_edge = edge_emb.shape[1]
    d_hid = wn.shape[0]
    d_out = wm.shape[0]
    grid = (n // block_n,)

    return pl.pallas_call(
        _agg_kernel,
        grid=grid,
        in_specs=[
            pl.BlockSpec((block_n, d_in), lambda i: (i, 0)),
            pl.BlockSpec((block_n * _N_SAMPLE, d_in), lambda i: (i, 0)),
            pl.BlockSpec(memory_space=pltpu.MemorySpace.HBM),
            pl.BlockSpec((d_hid, d_in), lambda i: (0, 0)),
            pl.BlockSpec((d_hid, d_in), lambda i: (0, 0)),
            pl.BlockSpec((d_hid, d_edge), lambda i: (0, 0)),
            pl.BlockSpec((d_out, d_hid), lambda i: (0, 0)),
            pl.BlockSpec((1, d_out), lambda i: (0, 0)),
        ],
        out_specs=pl.BlockSpec((block_n, d_out), lambda i: (i, 0)),
        out_shape=jax.ShapeDtypeStruct((n, d_out), jnp.float32),
        scratch_shapes=[
            pltpu.VMEM((2, block_n * _N_SAMPLE, d_edge), jnp.float32),
            pltpu.SemaphoreType.DMA((2, 4)),
        ],
        compiler_params=pltpu.CompilerParams(
            dimension_semantics=("arbitrary",)),
    )(x, neibs, edge_emb, wxn, wn, we, wm, b2d)


def kernel(x, neibs, edge_emb, mask, W_x, W_n, W_e, W_m, b_m):
    del mask
    n = x.shape[0]
    wxn = (W_x - W_n).astype(jnp.bfloat16)
    wn = W_n.astype(jnp.bfloat16)
    we = W_e.astype(jnp.bfloat16)
    wm = W_m.astype(jnp.bfloat16)
    b2d = b_m.reshape(1, -1)
    block_n = 400 if n % 400 == 0 else n
    return _run(x, neibs, edge_emb, wxn, wn, we, wm, b2d, block_n)
